# baseline (speedup 1.0000x reference)
import functools

import jax
import jax.numpy as jnp
from jax import lax
from jax.experimental import pallas as pl
from jax.experimental.pallas import tpu as pltpu

N = 10000
E = 160000
D_IN = 128
D_EDGE = 16
H = 32
OUT = 16
B = 64

NW = 32
CHUNK = 128
ROWS_PER_W = 40
EP = NW * ROWS_PER_W * CHUNK
NPAD = 10016
STRIPE = NPAD // 16

_BE = 4096


def _sigmoid(v):
    return 1.0 / (1.0 + jnp.exp(-v))


def _dense0_body(x_ref, w_ref, b_ref, o_ref):
    o_ref[...] = jnp.maximum(
        jnp.dot(x_ref[...], w_ref[...], preferred_element_type=jnp.float32)
        + b_ref[...], 0.0)


def _dense0(x, w, b2):
    return pl.pallas_call(
        _dense0_body,
        out_shape=jax.ShapeDtypeStruct((N, H), jnp.float32),
    )(x, w, b2)


def _msg_body(xj_ref, ea_ref, m2_ref, bm_ref, w1_ref, b1_ref, o_ref):
    xj = xj_ref[...]
    f = jnp.maximum(
        jnp.dot(ea_ref[...], w1_ref[...], preferred_element_type=jnp.float32)
        + b1_ref[...], 0.0)
    p = jnp.dot(xj, m2_ref[...], preferred_element_type=jnp.float32)
    acc = jnp.dot(xj, bm_ref[...], preferred_element_type=jnp.float32)
    for t in range(D_EDGE):
        acc = acc + f[:, t:t + 1] * p[:, t * H:(t + 1) * H]
    o_ref[...] = acc


def _msg(xj, ea, m2, bm, w1, b1):
    grid = EP // _BE
    return pl.pallas_call(
        _msg_body,
        grid=(grid,),
        in_specs=[
            pl.BlockSpec((_BE, H), lambda i: (i, 0)),
            pl.BlockSpec((_BE, D_EDGE), lambda i: (i, 0)),
            pl.BlockSpec((H, D_EDGE * H), lambda i: (0, 0)),
            pl.BlockSpec((H, H), lambda i: (0, 0)),
            pl.BlockSpec((D_EDGE, D_EDGE), lambda i: (0, 0)),
            pl.BlockSpec((1, D_EDGE), lambda i: (0, 0)),
        ],
        out_specs=pl.BlockSpec((_BE, H), lambda i: (i, 0)),
        out_shape=jax.ShapeDtypeStruct((EP, H), jnp.float32),
    )(xj, ea, m2, bm, w1, b1)


def _gru_body(agg2_ref, h_ref, wr_ref, bc_ref, wih_ref, bih_ref, whh_ref,
              bhh_ref, o_ref):
    h = h_ref[...]
    agg = agg2_ref[0] + agg2_ref[1]
    m = jnp.maximum(
        agg + jnp.dot(h, wr_ref[...], preferred_element_type=jnp.float32)
        + bc_ref[...], 0.0)
    gi = jnp.dot(m, wih_ref[...], preferred_element_type=jnp.float32) + bih_ref[...]
    gh = jnp.dot(h, whh_ref[...], preferred_element_type=jnp.float32) + bhh_ref[...]
    r = _sigmoid(gi[:, 0:H] + gh[:, 0:H])
    z = _sigmoid(gi[:, H:2 * H] + gh[:, H:2 * H])
    n = jnp.tanh(gi[:, 2 * H:3 * H] + r * gh[:, 2 * H:3 * H])
    o_ref[...] = (1.0 - z) * n + z * h


def _gru(agg2, h, wr, bc2, wih, bih2, whh, bhh2):
    return pl.pallas_call(
        _gru_body,
        out_shape=jax.ShapeDtypeStruct((NPAD, H), jnp.float32),
    )(agg2, h, wr, bc2, wih, bih2, whh, bhh2)


def _s2s_body(out_ref, batch_ref, wi0_ref, wh0_ref, bi0_ref, bh0_ref,
              wi1_ref, wh1_ref, bi1_ref, bh1_ref,
              wi2_ref, wh2_ref, bi2_ref, bh2_ref,
              wl1_ref, bl1_ref, wl2_ref, bl2_ref, y_ref):
    outs = out_ref[...]
    seg = batch_ref[...]
    ids = lax.broadcasted_iota(jnp.int32, (N, B), 1)
    onehot = (seg == ids).astype(jnp.float32)

    wis = (wi0_ref[...], wi1_ref[...], wi2_ref[...])
    whs = (wh0_ref[...], wh1_ref[...], wh2_ref[...])
    bis = (bi0_ref[...], bi1_ref[...], bi2_ref[...])
    bhs = (bh0_ref[...], bh1_ref[...], bh2_ref[...])

    q_star = jnp.zeros((B, 2 * H), jnp.float32)
    hx = [jnp.zeros((B, H), jnp.float32) for _ in range(3)]
    cx = [jnp.zeros((B, H), jnp.float32) for _ in range(3)]
    for _ in range(3):
        layer_in = q_star
        for l in range(3):
            gates = (jnp.dot(layer_in, wis[l], preferred_element_type=jnp.float32)
                     + bis[l]
                     + jnp.dot(hx[l], whs[l], preferred_element_type=jnp.float32)
                     + bhs[l])
            ig = _sigmoid(gates[:, 0:H])
            fg = _sigmoid(gates[:, H:2 * H])
            gg = jnp.tanh(gates[:, 2 * H:3 * H])
            og = _sigmoid(gates[:, 3 * H:4 * H])
            c_new = fg * cx[l] + ig * gg
            h_new = og * jnp.tanh(c_new)
            hx[l] = h_new
            cx[l] = c_new
            layer_in = h_new
        q = layer_in
        qb = jnp.dot(onehot, q, preferred_element_type=jnp.float32)
        e = jnp.sum(outs * qb, axis=1, keepdims=True)
        em = jnp.max(jnp.where(onehot > 0.0, e, -jnp.inf), axis=0,
                     keepdims=True)
        em = jnp.where(em > -3e38, em, 0.0)
        emb = jnp.dot(onehot, em.T, preferred_element_type=jnp.float32)
        a_num = jnp.exp(e - emb)
        denom = lax.dot_general(onehot, a_num,
                                (((0,), (0,)), ((), ())),
                                preferred_element_type=jnp.float32)
        denb = jnp.dot(onehot, denom, preferred_element_type=jnp.float32)
        a = a_num / denb
        r_vec = lax.dot_general(onehot, a * outs,
                                (((0,), (0,)), ((), ())),
                                preferred_element_type=jnp.float32)
        q_star = jnp.concatenate([q, r_vec], axis=1)
    y1 = jnp.maximum(
        jnp.dot(q_star, wl1_ref[...], preferred_element_type=jnp.float32)
        + bl1_ref[...], 0.0)
    y_ref[...] = (jnp.dot(y1, wl2_ref[...], preferred_element_type=jnp.float32)
                  + bl2_ref[...])


def _s2s(outs, batch2, args):
    return pl.pallas_call(
        _s2s_body,
        out_shape=jax.ShapeDtypeStruct((B, OUT), jnp.float32),
    )(outs, batch2, *args)


def kernel(x, edge_attr, edge_index, batch, W_lin0, b_lin0, W_nn1, b_nn1,
           W_nn2, b_nn2, W_root, b_conv, W_ih, b_ih, W_hh, b_hh,
           Wi0, Wh0, bi0, bh0, Wi1, Wh1, bi1, bh1, Wi2, Wh2, bi2, bh2,
           W_lin1, b_lin1, W_lin2, b_lin2):
    f32 = jnp.float32
    src = edge_index[0]
    dst = edge_index[1]
    srcp = jnp.concatenate([src, jnp.zeros((EP - E,), jnp.int32)])
    dstp = jnp.concatenate([dst, jnp.full((EP - E,), N, jnp.int32)])
    eap = jnp.concatenate([edge_attr, jnp.zeros((EP - E, D_EDGE), f32)])

    m2 = W_nn2.reshape(D_EDGE, H, H).transpose(1, 0, 2).reshape(H, D_EDGE * H)
    bm = b_nn2.reshape(H, H)

    out0 = _dense0(x, W_lin0, b_lin0.reshape(1, H))
    h = jnp.concatenate([out0, jnp.zeros((NPAD - N, H), f32)])

    for _ in range(3):
        xj = h[srcp]
        msg = _msg(xj, eap, m2, bm, W_nn1, b_nn1.reshape(1, D_EDGE))
        agg = jax.ops.segment_sum(msg, dstp, num_segments=NPAD)
        agg2 = jnp.stack([agg, jnp.zeros((NPAD, H), f32)])
        h = _gru(agg2, h, W_root, b_conv.reshape(1, H), W_ih.T,
                 b_ih.reshape(1, 3 * H), W_hh.T, b_hh.reshape(1, 3 * H))

    batch2 = batch.reshape(N, 1)
    s2s_args = (Wi0.T, Wh0.T, bi0.reshape(1, 4 * H), bh0.reshape(1, 4 * H),
                Wi1.T, Wh1.T, bi1.reshape(1, 4 * H), bh1.reshape(1, 4 * H),
                Wi2.T, Wh2.T, bi2.reshape(1, 4 * H), bh2.reshape(1, 4 * H),
                W_lin1, b_lin1.reshape(1, H), W_lin2, b_lin2.reshape(1, OUT))
    return _s2s(h[:N], batch2, s2s_args)


# speedup vs baseline: 1.7489x; 1.7489x over previous
import functools

import jax
import jax.numpy as jnp
from jax import lax
from jax.experimental import pallas as pl
from jax.experimental.pallas import tpu as pltpu
from jax.experimental.pallas import tpu_sc as plsc

N = 10000
E = 160000
D_IN = 128
D_EDGE = 16
H = 32
OUT = 16
B = 64

NW = 32
CHUNK = 128
ROWS_PER_W = 40
EP = NW * ROWS_PER_W * CHUNK
STRIPE = 632
NPAD = 16 * STRIPE
NROWS = NW * ROWS_PER_W

_BE = 4096

_SC_MESH = plsc.VectorSubcoreMesh(core_axis_name="c", subcore_axis_name="s",
                                  num_cores=2, num_subcores=16)


@functools.partial(
    pl.kernel, mesh=_SC_MESH,
    out_type=jax.ShapeDtypeStruct((EP, H), jnp.float32),
    compiler_params=pltpu.CompilerParams(use_tc_tiling_on_sc=False),
    scratch_types=[
        pltpu.VMEM((ROWS_PER_W, CHUNK), jnp.int32),
        pltpu.VMEM((CHUNK, H), jnp.float32),
        pltpu.SemaphoreType.DMA,
    ],
)
def _sc_gather(table_hbm, idx_hbm, out_hbm, idx_v, rows_v, sem):
    wid = lax.axis_index("s") * 2 + lax.axis_index("c")
    row0 = wid * ROWS_PER_W
    pltpu.sync_copy(idx_hbm.at[pl.ds(row0, ROWS_PER_W)], idx_v)

    def body(j, _):
        pltpu.async_copy(table_hbm.at[idx_v.at[j]], rows_v, sem).wait()
        pltpu.sync_copy(rows_v, out_hbm.at[pl.ds((row0 + j) * CHUNK, CHUNK)])
        return 0

    lax.fori_loop(0, ROWS_PER_W, body, 0, unroll=False)


@functools.partial(
    pl.kernel, mesh=_SC_MESH,
    out_type=jax.ShapeDtypeStruct((2 * NPAD, H), jnp.float32),
    compiler_params=pltpu.CompilerParams(use_tc_tiling_on_sc=False),
    scratch_types=[
        pltpu.VMEM((ROWS_PER_W, CHUNK), jnp.int32),
        pltpu.VMEM((CHUNK, H), jnp.float32),
        pltpu.VMEM_SHARED((NPAD, H), jnp.float32),
        pltpu.SemaphoreType.DMA,
    ],
)
def _sc_scatter(msg_hbm, idx_hbm, zeros_hbm, out_hbm, idx_v, rows_v, agg_sp,
                sem):
    c = lax.axis_index("c")
    s = lax.axis_index("s")
    wid = s * 2 + c
    pltpu.sync_copy(zeros_hbm.at[pl.ds(s * STRIPE, STRIPE)],
                    agg_sp.at[pl.ds(s * STRIPE, STRIPE)])
    plsc.subcore_barrier()
    row0 = wid * ROWS_PER_W
    pltpu.sync_copy(idx_hbm.at[pl.ds(row0, ROWS_PER_W)], idx_v)

    def body(j, _):
        pltpu.sync_copy(msg_hbm.at[pl.ds((row0 + j) * CHUNK, CHUNK)], rows_v)
        pltpu.sync_copy(rows_v, agg_sp.at[idx_v.at[j]], add=True)
        return 0

    lax.fori_loop(0, ROWS_PER_W, body, 0, unroll=False)
    plsc.subcore_barrier()
    pltpu.sync_copy(agg_sp.at[pl.ds(s * STRIPE, STRIPE)],
                    out_hbm.at[pl.ds(c * NPAD + s * STRIPE, STRIPE)])


def _sigmoid(v):
    return 1.0 / (1.0 + jnp.exp(-v))


def _dense0_body(x_ref, w_ref, b_ref, o_ref):
    o_ref[...] = jnp.maximum(
        jnp.dot(x_ref[...], w_ref[...], preferred_element_type=jnp.float32)
        + b_ref[...], 0.0)


def _dense0(x, w, b2):
    return pl.pallas_call(
        _dense0_body,
        out_shape=jax.ShapeDtypeStruct((N, H), jnp.float32),
    )(x, w, b2)


def _msg_body(xj_ref, ea_ref, m2_ref, bm_ref, w1_ref, b1_ref, o_ref):
    xj = xj_ref[...]
    f = jnp.maximum(
        jnp.dot(ea_ref[...], w1_ref[...], preferred_element_type=jnp.float32)
        + b1_ref[...], 0.0)
    p = jnp.dot(xj, m2_ref[...], preferred_element_type=jnp.float32)
    acc = jnp.dot(xj, bm_ref[...], preferred_element_type=jnp.float32)
    for t in range(D_EDGE):
        acc = acc + f[:, t:t + 1] * p[:, t * H:(t + 1) * H]
    o_ref[...] = acc


def _msg(xj, ea, m2, bm, w1, b1):
    grid = EP // _BE
    return pl.pallas_call(
        _msg_body,
        grid=(grid,),
        in_specs=[
            pl.BlockSpec((_BE, H), lambda i: (i, 0)),
            pl.BlockSpec((_BE, D_EDGE), lambda i: (i, 0)),
            pl.BlockSpec((H, D_EDGE * H), lambda i: (0, 0)),
            pl.BlockSpec((H, H), lambda i: (0, 0)),
            pl.BlockSpec((D_EDGE, D_EDGE), lambda i: (0, 0)),
            pl.BlockSpec((1, D_EDGE), lambda i: (0, 0)),
        ],
        out_specs=pl.BlockSpec((_BE, H), lambda i: (i, 0)),
        out_shape=jax.ShapeDtypeStruct((EP, H), jnp.float32),
    )(xj, ea, m2, bm, w1, b1)


def _gru_body(agg2_ref, h_ref, wr_ref, bc_ref, wih_ref, bih_ref, whh_ref,
              bhh_ref, o_ref):
    h = h_ref[...]
    agg = agg2_ref[0] + agg2_ref[1]
    m = jnp.maximum(
        agg + jnp.dot(h, wr_ref[...], preferred_element_type=jnp.float32)
        + bc_ref[...], 0.0)
    gi = jnp.dot(m, wih_ref[...], preferred_element_type=jnp.float32) + bih_ref[...]
    gh = jnp.dot(h, whh_ref[...], preferred_element_type=jnp.float32) + bhh_ref[...]
    r = _sigmoid(gi[:, 0:H] + gh[:, 0:H])
    z = _sigmoid(gi[:, H:2 * H] + gh[:, H:2 * H])
    n = jnp.tanh(gi[:, 2 * H:3 * H] + r * gh[:, 2 * H:3 * H])
    o_ref[...] = (1.0 - z) * n + z * h


def _gru(agg2, h, wr, bc2, wih, bih2, whh, bhh2):
    return pl.pallas_call(
        _gru_body,
        out_shape=jax.ShapeDtypeStruct((NPAD, H), jnp.float32),
    )(agg2, h, wr, bc2, wih, bih2, whh, bhh2)


def _s2s_body(out_ref, batch_ref, wi0_ref, wh0_ref, bi0_ref, bh0_ref,
              wi1_ref, wh1_ref, bi1_ref, bh1_ref,
              wi2_ref, wh2_ref, bi2_ref, bh2_ref,
              wl1_ref, bl1_ref, wl2_ref, bl2_ref, y_ref):
    outs = out_ref[...]
    seg = batch_ref[...]
    ids = lax.broadcasted_iota(jnp.int32, (N, B), 1)
    onehot = (seg == ids).astype(jnp.float32)

    wis = (wi0_ref[...], wi1_ref[...], wi2_ref[...])
    whs = (wh0_ref[...], wh1_ref[...], wh2_ref[...])
    bis = (bi0_ref[...], bi1_ref[...], bi2_ref[...])
    bhs = (bh0_ref[...], bh1_ref[...], bh2_ref[...])

    q_star = jnp.zeros((B, 2 * H), jnp.float32)
    hx = [jnp.zeros((B, H), jnp.float32) for _ in range(3)]
    cx = [jnp.zeros((B, H), jnp.float32) for _ in range(3)]
    for _ in range(3):
        layer_in = q_star
        for l in range(3):
            gates = (jnp.dot(layer_in, wis[l], preferred_element_type=jnp.float32)
                     + bis[l]
                     + jnp.dot(hx[l], whs[l], preferred_element_type=jnp.float32)
                     + bhs[l])
            ig = _sigmoid(gates[:, 0:H])
            fg = _sigmoid(gates[:, H:2 * H])
            gg = jnp.tanh(gates[:, 2 * H:3 * H])
            og = _sigmoid(gates[:, 3 * H:4 * H])
            c_new = fg * cx[l] + ig * gg
            h_new = og * jnp.tanh(c_new)
            hx[l] = h_new
            cx[l] = c_new
            layer_in = h_new
        q = layer_in
        qb = jnp.dot(onehot, q, preferred_element_type=jnp.float32)
        e = jnp.sum(outs * qb, axis=1, keepdims=True)
        em = jnp.max(jnp.where(onehot > 0.0, e, -jnp.inf), axis=0,
                     keepdims=True)
        em = jnp.where(em > -3e38, em, 0.0)
        emb = jnp.dot(onehot, em.T, preferred_element_type=jnp.float32)
        a_num = jnp.exp(e - emb)
        denom = lax.dot_general(onehot, a_num,
                                (((0,), (0,)), ((), ())),
                                preferred_element_type=jnp.float32)
        denb = jnp.dot(onehot, denom, preferred_element_type=jnp.float32)
        a = a_num / denb
        r_vec = lax.dot_general(onehot, a * outs,
                                (((0,), (0,)), ((), ())),
                                preferred_element_type=jnp.float32)
        q_star = jnp.concatenate([q, r_vec], axis=1)
    y1 = jnp.maximum(
        jnp.dot(q_star, wl1_ref[...], preferred_element_type=jnp.float32)
        + bl1_ref[...], 0.0)
    y_ref[...] = (jnp.dot(y1, wl2_ref[...], preferred_element_type=jnp.float32)
                  + bl2_ref[...])


def _s2s(outs, batch2, args):
    return pl.pallas_call(
        _s2s_body,
        out_shape=jax.ShapeDtypeStruct((B, OUT), jnp.float32),
    )(outs, batch2, *args)


def kernel(x, edge_attr, edge_index, batch, W_lin0, b_lin0, W_nn1, b_nn1,
           W_nn2, b_nn2, W_root, b_conv, W_ih, b_ih, W_hh, b_hh,
           Wi0, Wh0, bi0, bh0, Wi1, Wh1, bi1, bh1, Wi2, Wh2, bi2, bh2,
           W_lin1, b_lin1, W_lin2, b_lin2):
    f32 = jnp.float32
    src = edge_index[0]
    dst = edge_index[1]
    srcp = jnp.concatenate([src, jnp.zeros((EP - E,), jnp.int32)]
                           ).reshape(NROWS, CHUNK)
    dstp = jnp.concatenate([dst, jnp.full((EP - E,), N, jnp.int32)]
                           ).reshape(NROWS, CHUNK)
    eap = jnp.concatenate([edge_attr, jnp.zeros((EP - E, D_EDGE), f32)])
    zeros_npad = jnp.zeros((NPAD, H), f32)

    m2 = W_nn2.reshape(D_EDGE, H, H).transpose(1, 0, 2).reshape(H, D_EDGE * H)
    bm = b_nn2.reshape(H, H)

    out0 = _dense0(x, W_lin0, b_lin0.reshape(1, H))
    h = jnp.concatenate([out0, jnp.zeros((NPAD - N, H), f32)])

    for _ in range(3):
        xj = _sc_gather(h, srcp)
        msg = _msg(xj, eap, m2, bm, W_nn1, b_nn1.reshape(1, D_EDGE))
        agg2 = _sc_scatter(msg, dstp, zeros_npad).reshape(2, NPAD, H)
        h = _gru(agg2, h, W_root, b_conv.reshape(1, H), W_ih.T,
                 b_ih.reshape(1, 3 * H), W_hh.T, b_hh.reshape(1, 3 * H))

    batch2 = batch.reshape(N, 1)
    s2s_args = (Wi0.T, Wh0.T, bi0.reshape(1, 4 * H), bh0.reshape(1, 4 * H),
                Wi1.T, Wh1.T, bi1.reshape(1, 4 * H), bh1.reshape(1, 4 * H),
                Wi2.T, Wh2.T, bi2.reshape(1, 4 * H), bh2.reshape(1, 4 * H),
                W_lin1, b_lin1.reshape(1, H), W_lin2, b_lin2.reshape(1, OUT))
    return _s2s(h[:N], batch2, s2s_args)
